# baseline (speedup 1.0000x reference)
import functools

import jax
import jax.numpy as jnp
from jax import lax
from jax.experimental import pallas as pl
from jax.experimental.pallas import tpu as pltpu
from jax.experimental.pallas import tpu_sc as plsc

_N = 10000
_E = 320000

_NC = 2
_NS = 16
_NW = _NC * _NS
_EPT = _E // _NW
_C = 80
_NCHUNK = _EPT // _C
_NPAD = 10240
_RPW = _NPAD // _NS


def _edge_proj(edge_attr, W, b):
    K, D = W.shape
    BLK = 4000

    def body(ea_ref, w_ref, b_ref, o_ref):
        o_ref[...] = (
            jnp.dot(ea_ref[...], w_ref[...], preferred_element_type=jnp.float32)
            + b_ref[...]
        )

    return pl.pallas_call(
        body,
        grid=(_E // BLK,),
        in_specs=[
            pl.BlockSpec((BLK, K), lambda i: (i, 0)),
            pl.BlockSpec((K, D), lambda i: (0, 0)),
            pl.BlockSpec((1, D), lambda i: (0, 0)),
        ],
        out_specs=pl.BlockSpec((BLK, D), lambda i: (i, 0)),
        out_shape=jax.ShapeDtypeStruct((_E, D), jnp.float32),
    )(edge_attr, W, b.reshape(1, D))


def _make_msg(D):
    DV = D // 16
    mesh = plsc.VectorSubcoreMesh(core_axis_name="c", subcore_axis_name="s")

    @functools.partial(
        pl.kernel,
        out_type=jax.ShapeDtypeStruct((_NC, _NPAD, D), jnp.float32),
        mesh=mesh,
        scratch_types=[
            pltpu.VMEM((_C,), jnp.int32),
            pltpu.VMEM((_C,), jnp.int32),
            pltpu.VMEM((_C,), jnp.int32),
            pltpu.VMEM((_C,), jnp.int32),
            pltpu.VMEM((_C, D), jnp.float32),
            pltpu.VMEM((_C, D), jnp.float32),
            pltpu.VMEM((_C, 128), jnp.float32),
            pltpu.VMEM((_C, 128), jnp.float32),
            pltpu.VMEM_SHARED((_NPAD, D), jnp.float32),
            pltpu.SemaphoreType.DMA,
            pltpu.SemaphoreType.DMA,
            pltpu.SemaphoreType.DMA,
            pltpu.SemaphoreType.DMA,
            pltpu.SemaphoreType.DMA,
            pltpu.SemaphoreType.DMA,
            pltpu.SemaphoreType.DMA,
            pltpu.SemaphoreType.DMA,
            pltpu.SemaphoreType.DMA,
            pltpu.SemaphoreType.DMA,
        ],
    )
    def msg(x_hbm, e_hbm, src_hbm, dst_hbm, out_hbm,
            sib0, sib1, dib0, dib1, eb0, eb1, gb0, gb1, acc,
            sis0, sis1, dis0, dis1, es0, es1, gs0, gs1, ss0, ss1):
        cid = lax.axis_index("c")
        sid = lax.axis_index("s")
        wid = cid * _NS + sid
        sib = (sib0, sib1)
        dib = (dib0, dib1)
        eb = (eb0, eb1)
        gb = (gb0, gb1)
        sis = (sis0, sis1)
        dis = (dis0, dis1)
        es = (es0, es1)
        gs = (gs0, gs1)
        ss = (ss0, ss1)

        def base(c):
            return pl.multiple_of(wid * _EPT + c * _C, 8)

        def zrow(r, _):
            for d in range(DV):
                eb0[r, pl.ds(d * 16, 16)] = jnp.zeros((16,), jnp.float32)
            return 0
        lax.fori_loop(0, _C, zrow, 0)
        NZ = _RPW // _C
        for k in range(NZ):
            pltpu.async_copy(eb0, acc.at[pl.ds(sid * _RPW + k * _C, _C)], ss0)
        for k in range(NZ):
            pltpu.make_async_copy(eb0, acc.at[pl.ds(0, _C)], ss0).wait()
        plsc.subcore_barrier()

        def issue_src(c, p):
            pltpu.async_copy(src_hbm.at[pl.ds(base(c), _C)], sib[p], sis[p])

        def issue_dst(c, p):
            pltpu.async_copy(dst_hbm.at[pl.ds(base(c), _C)], dib[p], dis[p])

        def issue_e(c, p):
            pltpu.async_copy(e_hbm.at[pl.ds(base(c), _C)], eb[p], es[p])

        def issue_gather(p):
            pltpu.async_copy(x_hbm.at[sib[p]], gb[p], gs[p])

        def wait(buf, sem):
            pltpu.make_async_copy(e_hbm.at[pl.ds(0, _C)], buf, sem).wait()

        def wait_g(buf, sem):
            pltpu.make_async_copy(x_hbm.at[pl.ds(0, _C)], buf, sem).wait()

        def wait_idx(buf, sem):
            pltpu.make_async_copy(src_hbm.at[pl.ds(0, _C)], buf, sem).wait()

        def wait_scatter(p):
            pltpu.make_async_copy(eb[p], acc.at[pl.ds(0, _C)], ss[p]).wait()

        def compute(p):
            def row(r2, _):
                for rr in range(2):
                    r = 2 * r2 + rr
                    for d in range(DV):
                        sl = pl.ds(d * 16, 16)
                        eb[p][r, sl] = jnp.maximum(
                            gb[p][r, sl] + eb[p][r, sl], 0.0)
                return 0
            lax.fori_loop(0, _C // 2, row, 0)

        def body(c, p, first):
            q = 1 - p
            wait(eb[p], es[p])
            wait_g(gb[p], gs[p])

            @pl.when(c + 2 < _NCHUNK)
            def _():
                issue_src(c + 2, p)
            issue_dst(c, p)
            compute(p)
            wait_idx(dib[p], dis[p])
            if not first:
                wait_scatter(q)

            @pl.when(c + 1 < _NCHUNK)
            def _():
                issue_e(c + 1, q)
            pltpu.async_copy(eb[p], acc.at[dib[p]], ss[p], add=True)

            @pl.when(c + 2 < _NCHUNK)
            def _():
                wait_idx(sib[p], sis[p])
                issue_gather(p)

        issue_src(0, 0)
        issue_src(1, 1)
        issue_e(0, 0)
        for p in range(2):
            wait_idx(sib[p], sis[p])
            issue_gather(p)
        body(0, 0, first=True)
        body(1, 1, first=False)
        body(2, 0, first=False)

        def pair(i, _):
            body(2 * i + 3, 1, first=False)
            body(2 * i + 4, 0, first=False)
            return 0
        lax.fori_loop(0, (_NCHUNK - 3) // 2, pair, 0)

        wait_scatter(0)
        plsc.subcore_barrier()

        pltpu.sync_copy(
            acc.at[pl.ds(sid * _RPW, _RPW)],
            out_hbm.at[cid, pl.ds(sid * _RPW, _RPW)],
        )

    return msg


_msg128 = _make_msg(128)


def _post(x, parts, Wa, ba, g, t, Wb, bb, final_relu):
    Din = Wa.shape[0]
    Hh = Wa.shape[1]
    Do = Wb.shape[1]
    Dx = x.shape[1]
    Dp = parts.shape[2]

    def body(x_ref, p_ref, wa_ref, ba_ref, g_ref, t_ref, wb_ref, bb_ref, o_ref):
        h = x_ref[...][:, :Din] + p_ref[0][:, :Din] + p_ref[1][:, :Din]
        h1 = jnp.dot(h, wa_ref[...], preferred_element_type=jnp.float32) + ba_ref[...]
        mu = jnp.mean(h1, axis=0, keepdims=True)
        var = jnp.mean((h1 - mu) * (h1 - mu), axis=0, keepdims=True)
        h1 = (h1 - mu) * lax.rsqrt(var + 1e-5) * g_ref[...] + t_ref[...]
        h1 = jnp.maximum(h1, 0.0)
        out = jnp.dot(h1, wb_ref[...], preferred_element_type=jnp.float32) + bb_ref[...]
        if final_relu:
            out = jnp.maximum(out, 0.0)
        o_ref[...] = out

    return pl.pallas_call(
        body,
        grid=(1,),
        in_specs=[
            pl.BlockSpec((_N, Dx), lambda i: (0, 0)),
            pl.BlockSpec((_NC, _N, Dp), lambda i: (0, 0, 0)),
            pl.BlockSpec((Din, Hh), lambda i: (0, 0)),
            pl.BlockSpec((1, Hh), lambda i: (0, 0)),
            pl.BlockSpec((1, Hh), lambda i: (0, 0)),
            pl.BlockSpec((1, Hh), lambda i: (0, 0)),
            pl.BlockSpec((Hh, Do), lambda i: (0, 0)),
            pl.BlockSpec((1, Do), lambda i: (0, 0)),
        ],
        out_specs=pl.BlockSpec((_N, Do), lambda i: (0, 0)),
        out_shape=jax.ShapeDtypeStruct((_N, Do), jnp.float32),
    )(x, parts, Wa, ba.reshape(1, Hh), g.reshape(1, Hh), t.reshape(1, Hh),
      Wb, bb.reshape(1, Do))


def kernel(x, edge_index, edge_attr, We0, be0, W0a, b0a, g0, t0, W0b, b0b,
           We1, be1, W1a, b1a, g1, t1, W1b, b1b):
    src = edge_index[0]
    dst = edge_index[1]

    W0bp = jnp.pad(W0b, ((0, 0), (0, 64)))
    b0bp = jnp.pad(b0b, (0, 64))
    We1p = jnp.pad(We1, ((0, 0), (0, 64)))
    be1p = jnp.pad(be1, (0, 64))

    e0 = _edge_proj(edge_attr, We0, be0)
    e1 = _edge_proj(edge_attr, We1p, be1p)
    parts0 = _msg128(x, e0, src, dst)
    h = _post(x, parts0, W0a, b0a, g0, t0, W0bp, b0bp, final_relu=True)
    parts1 = _msg128(h, e1, src, dst)
    return _post(h, parts1, W1a, b1a, g1, t1, W1b, b1b, final_relu=False)


# speedup vs baseline: 1.1415x; 1.1415x over previous
import functools

import jax
import jax.numpy as jnp
from jax import lax
from jax.experimental import pallas as pl
from jax.experimental.pallas import tpu as pltpu
from jax.experimental.pallas import tpu_sc as plsc

_N = 10000
_E = 320000

_NC = 2
_NS = 16
_NW = _NC * _NS
_EPT = _E // _NW
_C = 40
_NCHUNK = _EPT // _C
_NPAD = 10240
_RPW = _NPAD // _NS


def _edge_proj(edge_attr, W, b):
    K, D = W.shape
    BLK = 4000

    def body(ea_ref, w_ref, b_ref, o_ref):
        o_ref[...] = (
            jnp.dot(ea_ref[...], w_ref[...], preferred_element_type=jnp.float32)
            + b_ref[...]
        )

    return pl.pallas_call(
        body,
        grid=(_E // BLK,),
        in_specs=[
            pl.BlockSpec((BLK, K), lambda i: (i, 0)),
            pl.BlockSpec((K, D), lambda i: (0, 0)),
            pl.BlockSpec((1, D), lambda i: (0, 0)),
        ],
        out_specs=pl.BlockSpec((BLK, D), lambda i: (i, 0)),
        out_shape=jax.ShapeDtypeStruct((_E, D), jnp.float32),
    )(edge_attr, W, b.reshape(1, D))


def _make_msg(D):
    DV = D // 16
    mesh = plsc.VectorSubcoreMesh(core_axis_name="c", subcore_axis_name="s")

    @functools.partial(
        pl.kernel,
        out_type=jax.ShapeDtypeStruct((_NC, _NPAD, D), jnp.float32),
        mesh=mesh,
        scratch_types=[
            pltpu.VMEM((_C,), jnp.int32),
            pltpu.VMEM((_C,), jnp.int32),
            pltpu.VMEM((_C,), jnp.int32),
            pltpu.VMEM((_C,), jnp.int32),
            pltpu.VMEM((_C, D), jnp.float32),
            pltpu.VMEM((_C, D), jnp.float32),
            pltpu.VMEM((_C, 128), jnp.float32),
            pltpu.VMEM((_C, 128), jnp.float32),
            pltpu.VMEM((_C, D), jnp.float32),
            pltpu.VMEM((_C, D), jnp.float32),
            pltpu.VMEM_SHARED((_NPAD, D), jnp.float32),
            pltpu.SemaphoreType.DMA,
            pltpu.SemaphoreType.DMA,
            pltpu.SemaphoreType.DMA,
            pltpu.SemaphoreType.DMA,
            pltpu.SemaphoreType.DMA,
            pltpu.SemaphoreType.DMA,
            pltpu.SemaphoreType.DMA,
            pltpu.SemaphoreType.DMA,
            pltpu.SemaphoreType.DMA,
            pltpu.SemaphoreType.DMA,
        ],
    )
    def msg(x_hbm, e_hbm, src_hbm, dst_hbm, out_hbm,
            sib0, sib1, dib0, dib1, eb0, eb1, gb0, gb1, mb0, mb1, acc,
            sis0, sis1, dis0, dis1, es0, es1, gs0, gs1, ss0, ss1):
        cid = lax.axis_index("c")
        sid = lax.axis_index("s")
        wid = cid * _NS + sid
        sib = (sib0, sib1)
        dib = (dib0, dib1)
        eb = (eb0, eb1)
        gb = (gb0, gb1)
        mb = (mb0, mb1)
        sis = (sis0, sis1)
        dis = (dis0, dis1)
        es = (es0, es1)
        gs = (gs0, gs1)
        ss = (ss0, ss1)

        def base(c):
            return pl.multiple_of(wid * _EPT + c * _C, 8)

        def zrow(r, _):
            for d in range(DV):
                mb0[r, pl.ds(d * 16, 16)] = jnp.zeros((16,), jnp.float32)
            return 0
        lax.fori_loop(0, _C, zrow, 0)
        NZ = _RPW // _C
        for k in range(NZ):
            pltpu.async_copy(mb0, acc.at[pl.ds(sid * _RPW + k * _C, _C)], ss0)
        for k in range(NZ):
            pltpu.make_async_copy(mb0, acc.at[pl.ds(0, _C)], ss0).wait()
        plsc.subcore_barrier()

        def issue_src(c, p):
            pltpu.async_copy(src_hbm.at[pl.ds(base(c), _C)], sib[p], sis[p])

        def issue_dst(c, p):
            pltpu.async_copy(dst_hbm.at[pl.ds(base(c), _C)], dib[p], dis[p])

        def issue_e(c, p):
            pltpu.async_copy(e_hbm.at[pl.ds(base(c), _C)], eb[p], es[p])

        def issue_gather(p):
            pltpu.async_copy(x_hbm.at[sib[p]], gb[p], gs[p])

        def wait(buf, sem):
            pltpu.make_async_copy(e_hbm.at[pl.ds(0, _C)], buf, sem).wait()

        def wait_g(buf, sem):
            pltpu.make_async_copy(x_hbm.at[pl.ds(0, _C)], buf, sem).wait()

        def wait_idx(buf, sem):
            pltpu.make_async_copy(src_hbm.at[pl.ds(0, _C)], buf, sem).wait()

        def wait_scatter(p):
            pltpu.make_async_copy(mb[p], acc.at[pl.ds(0, _C)], ss[p]).wait()

        def compute(p):
            def row(r2, _):
                for rr in range(2):
                    r = 2 * r2 + rr
                    for d in range(DV):
                        sl = pl.ds(d * 16, 16)
                        mb[p][r, sl] = jnp.maximum(
                            gb[p][r, sl] + eb[p][r, sl], 0.0)
                return 0
            lax.fori_loop(0, _C // 2, row, 0)

        def body(c, p, first):
            wait(eb[p], es[p])
            wait_g(gb[p], gs[p])

            @pl.when(c + 2 < _NCHUNK)
            def _():
                issue_src(c + 2, p)
            if not first:
                wait_scatter(p)
            issue_dst(c, p)
            compute(p)
            wait_idx(dib[p], dis[p])
            pltpu.async_copy(mb[p], acc.at[dib[p]], ss[p], add=True)

            @pl.when(c + 2 < _NCHUNK)
            def _():
                issue_e(c + 2, p)

            @pl.when(c + 2 < _NCHUNK)
            def _():
                wait_idx(sib[p], sis[p])
                issue_gather(p)

        for p in range(2):
            issue_src(p, p)
            issue_e(p, p)
        for p in range(2):
            wait_idx(sib[p], sis[p])
            issue_gather(p)
        body(0, 0, first=True)
        body(1, 1, first=True)

        def pair(i, _):
            body(2 * i + 2, 0, first=False)
            body(2 * i + 3, 1, first=False)
            return 0
        lax.fori_loop(0, (_NCHUNK - 2) // 2, pair, 0)

        wait_scatter(0)
        wait_scatter(1)
        plsc.subcore_barrier()

        pltpu.sync_copy(
            acc.at[pl.ds(sid * _RPW, _RPW)],
            out_hbm.at[cid, pl.ds(sid * _RPW, _RPW)],
        )

    return msg


_msg128 = _make_msg(128)


def _post(x, parts, Wa, ba, g, t, Wb, bb, final_relu):
    Din = Wa.shape[0]
    Hh = Wa.shape[1]
    Do = Wb.shape[1]
    Dx = x.shape[1]
    Dp = parts.shape[2]

    def body(x_ref, p_ref, wa_ref, ba_ref, g_ref, t_ref, wb_ref, bb_ref, o_ref):
        h = x_ref[...][:, :Din] + p_ref[0][:, :Din] + p_ref[1][:, :Din]
        h1 = jnp.dot(h, wa_ref[...], preferred_element_type=jnp.float32) + ba_ref[...]
        mu = jnp.mean(h1, axis=0, keepdims=True)
        var = jnp.mean((h1 - mu) * (h1 - mu), axis=0, keepdims=True)
        h1 = (h1 - mu) * lax.rsqrt(var + 1e-5) * g_ref[...] + t_ref[...]
        h1 = jnp.maximum(h1, 0.0)
        out = jnp.dot(h1, wb_ref[...], preferred_element_type=jnp.float32) + bb_ref[...]
        if final_relu:
            out = jnp.maximum(out, 0.0)
        o_ref[...] = out

    return pl.pallas_call(
        body,
        grid=(1,),
        in_specs=[
            pl.BlockSpec((_N, Dx), lambda i: (0, 0)),
            pl.BlockSpec((_NC, _N, Dp), lambda i: (0, 0, 0)),
            pl.BlockSpec((Din, Hh), lambda i: (0, 0)),
            pl.BlockSpec((1, Hh), lambda i: (0, 0)),
            pl.BlockSpec((1, Hh), lambda i: (0, 0)),
            pl.BlockSpec((1, Hh), lambda i: (0, 0)),
            pl.BlockSpec((Hh, Do), lambda i: (0, 0)),
            pl.BlockSpec((1, Do), lambda i: (0, 0)),
        ],
        out_specs=pl.BlockSpec((_N, Do), lambda i: (0, 0)),
        out_shape=jax.ShapeDtypeStruct((_N, Do), jnp.float32),
    )(x, parts, Wa, ba.reshape(1, Hh), g.reshape(1, Hh), t.reshape(1, Hh),
      Wb, bb.reshape(1, Do))


def kernel(x, edge_index, edge_attr, We0, be0, W0a, b0a, g0, t0, W0b, b0b,
           We1, be1, W1a, b1a, g1, t1, W1b, b1b):
    src = edge_index[0]
    dst = edge_index[1]

    W0bp = jnp.pad(W0b, ((0, 0), (0, 64)))
    b0bp = jnp.pad(b0b, (0, 64))
    We1p = jnp.pad(We1, ((0, 0), (0, 64)))
    be1p = jnp.pad(be1, (0, 64))

    e0 = _edge_proj(edge_attr, We0, be0)
    e1 = _edge_proj(edge_attr, We1p, be1p)
    parts0 = _msg128(x, e0, src, dst)
    h = _post(x, parts0, W0a, b0a, g0, t0, W0bp, b0bp, final_relu=True)
    parts1 = _msg128(h, e1, src, dst)
    return _post(h, parts1, W1a, b1a, g1, t1, W1b, b1b, final_relu=False)


# speedup vs baseline: 1.1444x; 1.0025x over previous
import functools

import jax
import jax.numpy as jnp
from jax import lax
from jax.experimental import pallas as pl
from jax.experimental.pallas import tpu as pltpu
from jax.experimental.pallas import tpu_sc as plsc

_N = 10000
_E = 320000

_NC = 2
_NS = 16
_NW = _NC * _NS
_EPT = _E // _NW
_C = 40
_NCHUNK = _EPT // _C
_NPAD = 10240
_RPW = _NPAD // _NS


def _edge_proj(edge_attr, W, b):
    K, D = W.shape
    BLK = 4000

    def body(ea_ref, w_ref, b_ref, o_ref):
        o_ref[...] = (
            jnp.dot(ea_ref[...], w_ref[...], preferred_element_type=jnp.float32)
            + b_ref[...]
        )

    return pl.pallas_call(
        body,
        grid=(_E // BLK,),
        in_specs=[
            pl.BlockSpec((BLK, K), lambda i: (i, 0)),
            pl.BlockSpec((K, D), lambda i: (0, 0)),
            pl.BlockSpec((1, D), lambda i: (0, 0)),
        ],
        out_specs=pl.BlockSpec((BLK, D), lambda i: (i, 0)),
        out_shape=jax.ShapeDtypeStruct((_E, D), jnp.float32),
    )(edge_attr, W, b.reshape(1, D))


def _make_msg(D):
    DV = D // 16
    mesh = plsc.VectorSubcoreMesh(core_axis_name="c", subcore_axis_name="s")

    @functools.partial(
        pl.kernel,
        out_type=jax.ShapeDtypeStruct((_NC, _NPAD, D), jnp.float32),
        mesh=mesh,
        scratch_types=[
            pltpu.VMEM((_C,), jnp.int32),
            pltpu.VMEM((_C,), jnp.int32),
            pltpu.VMEM((_C,), jnp.int32),
            pltpu.VMEM((_C,), jnp.int32),
            pltpu.VMEM((_C, D), jnp.float32),
            pltpu.VMEM((_C, D), jnp.float32),
            pltpu.VMEM((_C, 128), jnp.float32),
            pltpu.VMEM((_C, 128), jnp.float32),
            pltpu.VMEM((_C, D), jnp.float32),
            pltpu.VMEM((_C, D), jnp.float32),
            pltpu.VMEM_SHARED((_NPAD, D), jnp.float32),
            pltpu.SemaphoreType.DMA,
            pltpu.SemaphoreType.DMA,
            pltpu.SemaphoreType.DMA,
            pltpu.SemaphoreType.DMA,
            pltpu.SemaphoreType.DMA,
            pltpu.SemaphoreType.DMA,
            pltpu.SemaphoreType.DMA,
            pltpu.SemaphoreType.DMA,
            pltpu.SemaphoreType.DMA,
            pltpu.SemaphoreType.DMA,
        ],
    )
    def msg(x_hbm, e_hbm, src_hbm, dst_hbm, out_hbm,
            sib0, sib1, dib0, dib1, eb0, eb1, gb0, gb1, mb0, mb1, acc,
            sis0, sis1, dis0, dis1, es0, es1, gs0, gs1, ss0, ss1):
        cid = lax.axis_index("c")
        sid = lax.axis_index("s")
        wid = cid * _NS + sid
        sib = (sib0, sib1)
        dib = (dib0, dib1)
        eb = (eb0, eb1)
        gb = (gb0, gb1)
        mb = (mb0, mb1)
        sis = (sis0, sis1)
        dis = (dis0, dis1)
        es = (es0, es1)
        gs = (gs0, gs1)
        ss = (ss0, ss1)

        def base(c):
            return pl.multiple_of(wid * _EPT + c * _C, 8)

        def zrow(r, _):
            for d in range(DV):
                mb0[r, pl.ds(d * 16, 16)] = jnp.zeros((16,), jnp.float32)
            return 0
        lax.fori_loop(0, _C, zrow, 0)
        NZ = _RPW // _C
        for k in range(NZ):
            pltpu.async_copy(mb0, acc.at[pl.ds(sid * _RPW + k * _C, _C)], ss0)
        for k in range(NZ):
            pltpu.make_async_copy(mb0, acc.at[pl.ds(0, _C)], ss0).wait()
        plsc.subcore_barrier()

        def issue_src(c, p):
            pltpu.async_copy(src_hbm.at[pl.ds(base(c), _C)], sib[p], sis[p])

        def issue_dst(c, p):
            pltpu.async_copy(dst_hbm.at[pl.ds(base(c), _C)], dib[p], dis[p])

        def issue_e(c, p):
            pltpu.async_copy(e_hbm.at[pl.ds(base(c), _C)], eb[p], es[p])

        def issue_gather(p):
            pltpu.async_copy(x_hbm.at[sib[p]], gb[p], gs[p])

        def wait(buf, sem):
            pltpu.make_async_copy(e_hbm.at[pl.ds(0, _C)], buf, sem).wait()

        def wait_g(buf, sem):
            pltpu.make_async_copy(x_hbm.at[pl.ds(0, _C)], buf, sem).wait()

        def wait_idx(buf, sem):
            pltpu.make_async_copy(src_hbm.at[pl.ds(0, _C)], buf, sem).wait()

        def wait_scatter(p):
            pltpu.make_async_copy(mb[p], acc.at[pl.ds(0, _C)], ss[p]).wait()

        def compute(p):
            def row(r, _):
                for d in range(DV):
                    sl = pl.ds(d * 16, 16)
                    mb[p][r, sl] = jnp.maximum(gb[p][r, sl] + eb[p][r, sl], 0.0)
                return 0
            lax.fori_loop(0, _C, row, 0)

        def body(c, p, first):
            wait(eb[p], es[p])
            wait_g(gb[p], gs[p])

            @pl.when(c + 2 < _NCHUNK)
            def _():
                issue_src(c + 2, p)
            if not first:
                wait_scatter(p)
            issue_dst(c, p)
            compute(p)
            wait_idx(dib[p], dis[p])
            pltpu.async_copy(mb[p], acc.at[dib[p]], ss[p], add=True)

            @pl.when(c + 2 < _NCHUNK)
            def _():
                issue_e(c + 2, p)

            @pl.when(c + 2 < _NCHUNK)
            def _():
                wait_idx(sib[p], sis[p])
                issue_gather(p)

        for p in range(2):
            issue_src(p, p)
            issue_e(p, p)
        for p in range(2):
            wait_idx(sib[p], sis[p])
            issue_gather(p)
        body(0, 0, first=True)
        body(1, 1, first=True)

        def pair(i, _):
            body(2 * i + 2, 0, first=False)
            body(2 * i + 3, 1, first=False)
            return 0
        lax.fori_loop(0, (_NCHUNK - 2) // 2, pair, 0)

        wait_scatter(0)
        wait_scatter(1)
        plsc.subcore_barrier()

        pltpu.sync_copy(
            acc.at[pl.ds(sid * _RPW, _RPW)],
            out_hbm.at[cid, pl.ds(sid * _RPW, _RPW)],
        )

    return msg


_msg128 = _make_msg(128)


def _post(x, parts, Wa, ba, g, t, Wb, bb, final_relu):
    Din = Wa.shape[0]
    Hh = Wa.shape[1]
    Do = Wb.shape[1]
    Dx = x.shape[1]
    Dp = parts.shape[2]

    def body(x_ref, p_ref, wa_ref, ba_ref, g_ref, t_ref, wb_ref, bb_ref, o_ref):
        h = x_ref[...][:, :Din] + p_ref[0][:, :Din] + p_ref[1][:, :Din]
        h1 = jnp.dot(h, wa_ref[...], preferred_element_type=jnp.float32) + ba_ref[...]
        mu = jnp.mean(h1, axis=0, keepdims=True)
        var = jnp.mean((h1 - mu) * (h1 - mu), axis=0, keepdims=True)
        h1 = (h1 - mu) * lax.rsqrt(var + 1e-5) * g_ref[...] + t_ref[...]
        h1 = jnp.maximum(h1, 0.0)
        out = jnp.dot(h1, wb_ref[...], preferred_element_type=jnp.float32) + bb_ref[...]
        if final_relu:
            out = jnp.maximum(out, 0.0)
        o_ref[...] = out

    return pl.pallas_call(
        body,
        grid=(1,),
        in_specs=[
            pl.BlockSpec((_N, Dx), lambda i: (0, 0)),
            pl.BlockSpec((_NC, _N, Dp), lambda i: (0, 0, 0)),
            pl.BlockSpec((Din, Hh), lambda i: (0, 0)),
            pl.BlockSpec((1, Hh), lambda i: (0, 0)),
            pl.BlockSpec((1, Hh), lambda i: (0, 0)),
            pl.BlockSpec((1, Hh), lambda i: (0, 0)),
            pl.BlockSpec((Hh, Do), lambda i: (0, 0)),
            pl.BlockSpec((1, Do), lambda i: (0, 0)),
        ],
        out_specs=pl.BlockSpec((_N, Do), lambda i: (0, 0)),
        out_shape=jax.ShapeDtypeStruct((_N, Do), jnp.float32),
    )(x, parts, Wa, ba.reshape(1, Hh), g.reshape(1, Hh), t.reshape(1, Hh),
      Wb, bb.reshape(1, Do))


def kernel(x, edge_index, edge_attr, We0, be0, W0a, b0a, g0, t0, W0b, b0b,
           We1, be1, W1a, b1a, g1, t1, W1b, b1b):
    src = edge_index[0]
    dst = edge_index[1]

    W0bp = jnp.pad(W0b, ((0, 0), (0, 64)))
    b0bp = jnp.pad(b0b, (0, 64))
    We1p = jnp.pad(We1, ((0, 0), (0, 64)))
    be1p = jnp.pad(be1, (0, 64))

    e0 = _edge_proj(edge_attr, We0, be0)
    e1 = _edge_proj(edge_attr, We1p, be1p)
    parts0 = _msg128(x, e0, src, dst)
    h = _post(x, parts0, W0a, b0a, g0, t0, W0bp, b0bp, final_relu=True)
    parts1 = _msg128(h, e1, src, dst)
    return _post(h, parts1, W1a, b1a, g1, t1, W1b, b1b, final_relu=False)
